# Initial kernel scaffold; baseline (speedup 1.0000x reference)
import jax
import jax.numpy as jnp
from jax.experimental import pallas as pl


def kernel(_paragraphs, features, table):
    raise NotImplementedError("write your pallas kernel here")


# speedup vs baseline: 1.0843x; 1.0843x over previous
import functools

import jax
import jax.numpy as jnp
from jax import lax
from jax.experimental import pallas as pl
from jax.experimental.pallas import tpu as pltpu
from jax.experimental.pallas import tpu_sc as plsc

_NC = 2
_NS = 16
_LANES = 16


def _row_means(table):
    v, e = table.shape
    grid = 125
    rb = v // grid
    assert rb * grid == v

    def body(x_ref, o_ref):
        x = x_ref[0]
        o_ref[0, 0, :] = jnp.sum(x, axis=-1) * (1.0 / e)

    out = pl.pallas_call(
        body,
        grid=(grid,),
        in_specs=[pl.BlockSpec((1, rb, e), lambda i: (i, 0, 0))],
        out_specs=pl.BlockSpec((1, 1, rb), lambda i: (i, 0, 0)),
        out_shape=jax.ShapeDtypeStruct((grid, 1, rb), jnp.float32),
    )(table.reshape(grid, rb, e))
    return out.reshape(v)


def _gather_softmax(features, means):
    b, _, l = features.shape
    nw = _NC * _NS
    bw = b // nw
    assert bw * nw == b
    c0 = (l // 2 + 7) // 8 * 8
    c1 = l - c0
    lp = (l + _LANES - 1) // _LANES * _LANES
    nk = lp // _LANES
    tail = l - (nk - 1) * _LANES

    mesh = plsc.VectorSubcoreMesh(core_axis_name="c", subcore_axis_name="s")

    @functools.partial(
        pl.kernel,
        mesh=mesh,
        out_type=jax.ShapeDtypeStruct((b, l), jnp.float32),
        scratch_types=[
            pltpu.VMEM((bw, l), jnp.int32),
            pltpu.VMEM((bw, lp), jnp.float32),
            pltpu.SemaphoreType.DMA,
        ],
        compiler_params=pltpu.CompilerParams(
            use_tc_tiling_on_sc=False, needs_layout_passes=False
        ),
    )
    def kern(feat_hbm, means_hbm, out_hbm, idx_v, vals_v, gsem):
        wid = lax.axis_index("s") * _NC + lax.axis_index("c")
        b0 = wid * bw

        pltpu.sync_copy(feat_hbm.at[pl.ds(b0, bw), 0, :], idx_v)

        def fire(j, carry):
            pltpu.async_copy(
                means_hbm.at[idx_v.at[j, pl.ds(0, c0)]],
                vals_v.at[j, pl.ds(0, c0)],
                gsem,
            )
            pltpu.async_copy(
                means_hbm.at[idx_v.at[j, pl.ds(c0, c1)]],
                vals_v.at[j, pl.ds(c0, c1)],
                gsem,
            )
            return carry

        lax.fori_loop(0, bw, fire, 0)

        pltpu.make_async_copy(
            out_hbm.at[pl.ds(b0, bw), :],
            vals_v.at[:, pl.ds(0, l)],
            gsem,
        ).wait()

        lane = lax.iota(jnp.int32, _LANES)
        tail_mask = lane < tail
        neg_inf = jnp.full((_LANES,), -jnp.inf, jnp.float32)

        def softmax_row(j, carry):
            vs = []
            for k in range(nk):
                v = vals_v[j, pl.ds(k * _LANES, _LANES)]
                if k == nk - 1:
                    v = jnp.where(tail_mask, v, neg_inf)
                vs.append(v)
            m = vs[0]
            for k in range(1, nk):
                m = jnp.maximum(m, vs[k])
            mb = jnp.broadcast_to(jnp.max(m), (_LANES,))
            es = []
            s = jnp.zeros((_LANES,), jnp.float32)
            for k in range(nk):
                ev = jnp.exp(vs[k] - mb)
                if k == nk - 1:
                    ev = jnp.where(tail_mask, ev, 0.0)
                es.append(ev)
                s = s + ev
            sb = jnp.broadcast_to(jnp.sum(s), (_LANES,))
            inv = 1.0 / sb
            for k in range(nk):
                vals_v[j, pl.ds(k * _LANES, _LANES)] = es[k] * inv
            return carry

        lax.fori_loop(0, bw, softmax_row, 0)

        pltpu.sync_copy(vals_v.at[:, pl.ds(0, l)], out_hbm.at[pl.ds(b0, bw), :])

    return kern(features, means)


def kernel(_paragraphs, features, table):
    means = _row_means(table)
    out = _gather_softmax(features, means)
    return out[:, None, :]


# speedup vs baseline: 2.1429x; 1.9763x over previous
import functools

import jax
import jax.numpy as jnp
from jax import lax
from jax.experimental import pallas as pl
from jax.experimental.pallas import tpu as pltpu
from jax.experimental.pallas import tpu_sc as plsc

_NC = 2
_NS = 16
_LANES = 16


def _row_means(table):
    v, e = table.shape
    rb = 16384
    t = rb // e
    grid = (v + rb - 1) // rb

    def body(x_ref, o_ref):
        x = x_ref[...].reshape(t, e, e)
        xt = jnp.transpose(x, (0, 2, 1))
        o_ref[...] = jnp.sum(xt, axis=1) * (1.0 / e)

    out = pl.pallas_call(
        body,
        grid=(grid,),
        in_specs=[pl.BlockSpec((rb, e), lambda i: (i, 0))],
        out_specs=pl.BlockSpec((t, e), lambda i: (i, 0)),
        out_shape=jax.ShapeDtypeStruct((grid * t, e), jnp.float32),
    )(table)
    return out.reshape(grid * rb)[:v]


def _gather_softmax(features, means):
    b, _, l = features.shape
    nw = _NC * _NS
    bw = b // nw
    assert bw * nw == b
    c0 = (l // 2 + 7) // 8 * 8
    c1 = l - c0
    lp = (l + _LANES - 1) // _LANES * _LANES
    nk = lp // _LANES
    tail = l - (nk - 1) * _LANES

    mesh = plsc.VectorSubcoreMesh(core_axis_name="c", subcore_axis_name="s")

    @functools.partial(
        pl.kernel,
        mesh=mesh,
        out_type=jax.ShapeDtypeStruct((b, l), jnp.float32),
        scratch_types=[
            pltpu.VMEM((bw, l), jnp.int32),
            pltpu.VMEM((bw, lp), jnp.float32),
            pltpu.SemaphoreType.DMA,
        ],
        compiler_params=pltpu.CompilerParams(
            use_tc_tiling_on_sc=False, needs_layout_passes=False
        ),
    )
    def kern(feat_hbm, means_hbm, out_hbm, idx_v, vals_v, gsem):
        wid = lax.axis_index("s") * _NC + lax.axis_index("c")
        b0 = wid * bw

        pltpu.sync_copy(feat_hbm.at[pl.ds(b0, bw), 0, :], idx_v)

        def fire(j, carry):
            pltpu.async_copy(
                means_hbm.at[idx_v.at[j, pl.ds(0, c0)]],
                vals_v.at[j, pl.ds(0, c0)],
                gsem,
            )
            pltpu.async_copy(
                means_hbm.at[idx_v.at[j, pl.ds(c0, c1)]],
                vals_v.at[j, pl.ds(c0, c1)],
                gsem,
            )
            return carry

        lax.fori_loop(0, bw, fire, 0)

        pltpu.make_async_copy(
            out_hbm.at[pl.ds(b0, bw), :],
            vals_v.at[:, pl.ds(0, l)],
            gsem,
        ).wait()

        lane = lax.iota(jnp.int32, _LANES)
        tail_mask = lane < tail
        neg_inf = jnp.full((_LANES,), -jnp.inf, jnp.float32)

        def softmax_row(j, carry):
            vs = []
            for k in range(nk):
                v = vals_v[j, pl.ds(k * _LANES, _LANES)]
                if k == nk - 1:
                    v = jnp.where(tail_mask, v, neg_inf)
                vs.append(v)
            m = vs[0]
            for k in range(1, nk):
                m = jnp.maximum(m, vs[k])
            mb = jnp.broadcast_to(jnp.max(m), (_LANES,))
            es = []
            s = jnp.zeros((_LANES,), jnp.float32)
            for k in range(nk):
                ev = jnp.exp(vs[k] - mb)
                if k == nk - 1:
                    ev = jnp.where(tail_mask, ev, 0.0)
                es.append(ev)
                s = s + ev
            sb = jnp.broadcast_to(jnp.sum(s), (_LANES,))
            inv = 1.0 / sb
            for k in range(nk):
                vals_v[j, pl.ds(k * _LANES, _LANES)] = es[k] * inv
            return carry

        lax.fori_loop(0, bw, softmax_row, 0)

        pltpu.sync_copy(vals_v.at[:, pl.ds(0, l)], out_hbm.at[pl.ds(b0, bw), :])

    return kern(features, means)


def kernel(_paragraphs, features, table):
    means = _row_means(table)
    out = _gather_softmax(features, means)
    return out[:, None, :]


# speedup vs baseline: 2.1948x; 1.0242x over previous
import functools

import jax
import jax.numpy as jnp
from jax import lax
from jax.experimental import pallas as pl
from jax.experimental.pallas import tpu as pltpu
from jax.experimental.pallas import tpu_sc as plsc

_NC = 2
_NS = 16
_LANES = 16


def _row_means(table):
    v, e = table.shape
    rb = 32768
    t = rb // e
    grid = (v + rb - 1) // rb

    def body(x_ref, o_ref):
        x = x_ref[...].reshape(t, e, e)
        xt = jnp.transpose(x, (0, 2, 1))
        o_ref[...] = jnp.sum(xt, axis=1) * (1.0 / e)

    out = pl.pallas_call(
        body,
        grid=(grid,),
        in_specs=[pl.BlockSpec((rb, e), lambda i: (i, 0))],
        out_specs=pl.BlockSpec((t, e), lambda i: (i, 0)),
        out_shape=jax.ShapeDtypeStruct((grid * t, e), jnp.float32),
    )(table)
    return out.reshape(grid * rb)


def _gather_softmax(features, means):
    b, _, l = features.shape
    nw = _NC * _NS
    bw = b // nw
    assert bw * nw == b
    c0 = (l // 2 + 7) // 8 * 8
    c1 = l - c0
    lp = (l + _LANES - 1) // _LANES * _LANES
    nk = lp // _LANES
    tail = l - (nk - 1) * _LANES

    mesh = plsc.VectorSubcoreMesh(core_axis_name="c", subcore_axis_name="s")

    @functools.partial(
        pl.kernel,
        mesh=mesh,
        out_type=jax.ShapeDtypeStruct((b, 1, l), jnp.float32),
        scratch_types=[
            pltpu.VMEM((bw, l), jnp.int32),
            pltpu.VMEM((bw, lp), jnp.float32),
            pltpu.SemaphoreType.DMA,
        ],
        compiler_params=pltpu.CompilerParams(
            use_tc_tiling_on_sc=False, needs_layout_passes=False
        ),
    )
    def kern(feat_hbm, means_hbm, out_hbm, idx_v, vals_v, gsem):
        wid = lax.axis_index("s") * _NC + lax.axis_index("c")
        b0 = wid * bw

        pltpu.sync_copy(feat_hbm.at[pl.ds(b0, bw), 0, :], idx_v)

        def fire(j, carry):
            pltpu.async_copy(
                means_hbm.at[idx_v.at[j, pl.ds(0, c0)]],
                vals_v.at[j, pl.ds(0, c0)],
                gsem,
            )
            pltpu.async_copy(
                means_hbm.at[idx_v.at[j, pl.ds(c0, c1)]],
                vals_v.at[j, pl.ds(c0, c1)],
                gsem,
            )
            return carry

        lax.fori_loop(0, bw, fire, 0)

        pltpu.make_async_copy(
            out_hbm.at[pl.ds(b0, bw), 0, :],
            vals_v.at[:, pl.ds(0, l)],
            gsem,
        ).wait()

        lane = lax.iota(jnp.int32, _LANES)
        tail_mask = lane < tail
        neg_inf = jnp.full((_LANES,), -jnp.inf, jnp.float32)

        def softmax_row(j, carry):
            vs = []
            for k in range(nk):
                v = vals_v[j, pl.ds(k * _LANES, _LANES)]
                if k == nk - 1:
                    v = jnp.where(tail_mask, v, neg_inf)
                vs.append(v)
            m = vs[0]
            for k in range(1, nk):
                m = jnp.maximum(m, vs[k])
            mb = jnp.broadcast_to(jnp.max(m), (_LANES,))
            es = []
            s = jnp.zeros((_LANES,), jnp.float32)
            for k in range(nk):
                ev = jnp.exp(vs[k] - mb)
                if k == nk - 1:
                    ev = jnp.where(tail_mask, ev, 0.0)
                es.append(ev)
                s = s + ev
            sb = jnp.broadcast_to(jnp.sum(s), (_LANES,))
            inv = 1.0 / sb
            for k in range(nk):
                vals_v[j, pl.ds(k * _LANES, _LANES)] = es[k] * inv
            return carry

        lax.fori_loop(0, bw, softmax_row, 0)

        pltpu.sync_copy(
            vals_v.at[:, pl.ds(0, l)], out_hbm.at[pl.ds(b0, bw), 0, :]
        )

    return kern(features, means)


def kernel(_paragraphs, features, table):
    means = _row_means(table)
    return _gather_softmax(features, means)


# speedup vs baseline: 2.2050x; 1.0046x over previous
import functools

import jax
import jax.numpy as jnp
from jax import lax
from jax.experimental import pallas as pl
from jax.experimental.pallas import tpu as pltpu
from jax.experimental.pallas import tpu_sc as plsc

_NC = 2
_NS = 16
_LANES = 16


def _row_means(table):
    v, e = table.shape
    rb = 32768
    t = rb // e
    grid = (v + rb - 1) // rb

    def body(x_ref, o_ref):
        x = x_ref[...].reshape(t, e, e)
        xt = jnp.transpose(x, (0, 2, 1))
        s = jnp.sum(xt, axis=1) * (1.0 / e)
        o_ref[...] = s.reshape(rb)

    out = pl.pallas_call(
        body,
        grid=(grid,),
        in_specs=[pl.BlockSpec((rb, e), lambda i: (i, 0))],
        out_specs=pl.BlockSpec((rb,), lambda i: (i,)),
        out_shape=jax.ShapeDtypeStruct((grid * rb,), jnp.float32),
    )(table)
    return out


def _gather_softmax(features, means):
    b, _, l = features.shape
    nw = _NC * _NS
    bw = b // nw
    assert bw * nw == b
    c0 = (l // 2 + 7) // 8 * 8
    c1 = l - c0
    lp = (l + _LANES - 1) // _LANES * _LANES
    nk = lp // _LANES
    tail = l - (nk - 1) * _LANES

    mesh = plsc.VectorSubcoreMesh(core_axis_name="c", subcore_axis_name="s")

    @functools.partial(
        pl.kernel,
        mesh=mesh,
        out_type=jax.ShapeDtypeStruct((b, 1, l), jnp.float32),
        scratch_types=[
            pltpu.VMEM((bw, l), jnp.int32),
            pltpu.VMEM((bw, lp), jnp.float32),
            pltpu.SemaphoreType.DMA,
        ],
        compiler_params=pltpu.CompilerParams(
            use_tc_tiling_on_sc=False, needs_layout_passes=False
        ),
    )
    def kern(feat_hbm, means_hbm, out_hbm, idx_v, vals_v, gsem):
        mflat = means_hbm
        wid = lax.axis_index("s") * _NC + lax.axis_index("c")
        b0 = wid * bw

        pltpu.sync_copy(feat_hbm.at[pl.ds(b0, bw), 0, :], idx_v)

        def fire(j, carry):
            pltpu.async_copy(
                mflat.at[idx_v.at[j, pl.ds(0, c0)]],
                vals_v.at[j, pl.ds(0, c0)],
                gsem,
            )
            pltpu.async_copy(
                mflat.at[idx_v.at[j, pl.ds(c0, c1)]],
                vals_v.at[j, pl.ds(c0, c1)],
                gsem,
            )
            return carry

        lax.fori_loop(0, bw, fire, 0)

        pltpu.make_async_copy(
            out_hbm.at[pl.ds(b0, bw), 0, :],
            vals_v.at[:, pl.ds(0, l)],
            gsem,
        ).wait()

        lane = lax.iota(jnp.int32, _LANES)
        tail_mask = lane < tail
        neg_inf = jnp.full((_LANES,), -jnp.inf, jnp.float32)

        def softmax_row(j, carry):
            vs = []
            for k in range(nk):
                v = vals_v[j, pl.ds(k * _LANES, _LANES)]
                if k == nk - 1:
                    v = jnp.where(tail_mask, v, neg_inf)
                vs.append(v)
            m = vs[0]
            for k in range(1, nk):
                m = jnp.maximum(m, vs[k])
            mb = jnp.broadcast_to(jnp.max(m), (_LANES,))
            es = []
            s = jnp.zeros((_LANES,), jnp.float32)
            for k in range(nk):
                ev = jnp.exp(vs[k] - mb)
                if k == nk - 1:
                    ev = jnp.where(tail_mask, ev, 0.0)
                es.append(ev)
                s = s + ev
            sb = jnp.broadcast_to(jnp.sum(s), (_LANES,))
            inv = 1.0 / sb
            for k in range(nk):
                vals_v[j, pl.ds(k * _LANES, _LANES)] = es[k] * inv
            return carry

        lax.fori_loop(0, bw, softmax_row, 0)

        pltpu.sync_copy(
            vals_v.at[:, pl.ds(0, l)], out_hbm.at[pl.ds(b0, bw), 0, :]
        )

    return kern(features, means)


def kernel(_paragraphs, features, table):
    means = _row_means(table)
    return _gather_softmax(features, means)


# speedup vs baseline: 2.3775x; 1.0783x over previous
import functools

import jax
import jax.numpy as jnp
from jax import lax
from jax.experimental import pallas as pl
from jax.experimental.pallas import tpu as pltpu
from jax.experimental.pallas import tpu_sc as plsc

_NC = 2
_NS = 16
_LANES = 16


def _row_means(table):
    v, e = table.shape
    rb = 32768
    t = rb // e
    grid = (v + rb - 1) // rb

    def body(x_ref, o_ref):
        x = x_ref[...].reshape(t, e, e)
        xt = jnp.transpose(x, (0, 2, 1))
        s = jnp.sum(xt, axis=1) * (1.0 / e)
        o_ref[...] = s.reshape(rb)

    out = pl.pallas_call(
        body,
        grid=(grid,),
        in_specs=[pl.BlockSpec((rb, e), lambda i: (i, 0))],
        out_specs=pl.BlockSpec((rb,), lambda i: (i,)),
        out_shape=jax.ShapeDtypeStruct((grid * rb,), jnp.float32),
    )(table)
    return out


def _gather_softmax(feats_flat, means, b, l):
    nw = _NC * _NS
    bw = b // nw
    assert bw * nw == b
    c0 = (l // 2 + 7) // 8 * 8
    c1 = l - c0
    lp = (l + _LANES - 1) // _LANES * _LANES
    nk = lp // _LANES
    tail = l - (nk - 1) * _LANES

    mesh = plsc.VectorSubcoreMesh(core_axis_name="c", subcore_axis_name="s")

    @functools.partial(
        pl.kernel,
        mesh=mesh,
        out_type=jax.ShapeDtypeStruct((b, 1, l), jnp.float32),
        scratch_types=[
            pltpu.VMEM((bw * l,), jnp.int32),
            pltpu.VMEM((bw, lp), jnp.float32),
            pltpu.SemaphoreType.DMA,
        ],
        compiler_params=pltpu.CompilerParams(
            use_tc_tiling_on_sc=False, needs_layout_passes=False
        ),
    )
    def kern(feat_hbm, means_hbm, out_hbm, idx_v, vals_v, gsem):
        mflat = means_hbm
        wid = lax.axis_index("s") * _NC + lax.axis_index("c")
        b0 = wid * bw

        pltpu.sync_copy(feat_hbm.at[pl.ds(b0 * l, bw * l)], idx_v)

        def fire(j, carry):
            pltpu.async_copy(
                mflat.at[idx_v.at[pl.ds(j * l, c0)]],
                vals_v.at[j, pl.ds(0, c0)],
                gsem,
            )
            pltpu.async_copy(
                mflat.at[idx_v.at[pl.ds(j * l + c0, c1)]],
                vals_v.at[j, pl.ds(c0, c1)],
                gsem,
            )
            return carry

        lax.fori_loop(0, bw, fire, 0)

        pltpu.make_async_copy(
            out_hbm.at[pl.ds(b0, bw), 0, :],
            vals_v.at[:, pl.ds(0, l)],
            gsem,
        ).wait()

        lane = lax.iota(jnp.int32, _LANES)
        tail_mask = lane < tail
        neg_inf = jnp.full((_LANES,), -jnp.inf, jnp.float32)

        def softmax_row(j, carry):
            vs = []
            for k in range(nk):
                v = vals_v[j, pl.ds(k * _LANES, _LANES)]
                if k == nk - 1:
                    v = jnp.where(tail_mask, v, neg_inf)
                vs.append(v)
            m = vs[0]
            for k in range(1, nk):
                m = jnp.maximum(m, vs[k])
            mb = jnp.broadcast_to(jnp.max(m), (_LANES,))
            es = []
            s = jnp.zeros((_LANES,), jnp.float32)
            for k in range(nk):
                ev = jnp.exp(vs[k] - mb)
                if k == nk - 1:
                    ev = jnp.where(tail_mask, ev, 0.0)
                es.append(ev)
                s = s + ev
            sb = jnp.broadcast_to(jnp.sum(s), (_LANES,))
            inv = 1.0 / sb
            for k in range(nk):
                vals_v[j, pl.ds(k * _LANES, _LANES)] = es[k] * inv
            return carry

        lax.fori_loop(0, bw, softmax_row, 0)

        pltpu.sync_copy(
            vals_v.at[:, pl.ds(0, l)], out_hbm.at[pl.ds(b0, bw), 0, :]
        )

    return kern(feats_flat, means)


def kernel(_paragraphs, features, table):
    b, _, l = features.shape
    means = _row_means(table)
    feats_flat = features[:, 0, :].reshape(b * l)
    return _gather_softmax(feats_flat, means, b, l)


# speedup vs baseline: 2.6989x; 1.1352x over previous
import functools

import jax
import jax.numpy as jnp
from jax import lax
from jax.experimental import pallas as pl
from jax.experimental.pallas import tpu as pltpu
from jax.experimental.pallas import tpu_sc as plsc

_NC = 2
_NS = 16
_LANES = 16


def _row_means(table):
    v, e = table.shape
    rb = 32768
    t = rb // e
    grid = (v + rb - 1) // rb

    def body(x_ref, o_ref):
        x = x_ref[...].reshape(t, e, e)
        xt = jnp.transpose(x, (0, 2, 1))
        s = jnp.sum(xt, axis=1) * (1.0 / e)
        o_ref[...] = s.reshape(rb)

    out = pl.pallas_call(
        body,
        grid=(grid,),
        in_specs=[pl.BlockSpec((rb, e), lambda i: (i, 0))],
        out_specs=pl.BlockSpec((rb,), lambda i: (i,)),
        out_shape=jax.ShapeDtypeStruct((grid * rb,), jnp.float32),
    )(table)
    return out


def _gather_softmax_t(feats_t, means, b, l):
    nw = _NC * _NS
    bw = b // nw
    assert bw * nw == b
    assert bw <= 128
    ngr = bw // _LANES
    assert ngr * _LANES == bw

    mesh = plsc.VectorSubcoreMesh(core_axis_name="c", subcore_axis_name="s")

    @functools.partial(
        pl.kernel,
        mesh=mesh,
        out_type=jax.ShapeDtypeStruct((l, b), jnp.float32),
        scratch_types=[
            pltpu.VMEM((l, bw), jnp.int32),
            pltpu.VMEM((l, bw), jnp.float32),
            pltpu.SemaphoreType.DMA,
        ],
        compiler_params=pltpu.CompilerParams(
            use_tc_tiling_on_sc=False, needs_layout_passes=False
        ),
    )
    def kern(feat_hbm, means_hbm, out_hbm, idx_v, vals_v, gsem):
        wid = lax.axis_index("s") * _NC + lax.axis_index("c")
        b0 = wid * bw

        pltpu.sync_copy(feat_hbm.at[:, pl.ds(b0, bw)], idx_v)

        def fire(li, carry):
            pltpu.async_copy(
                means_hbm.at[idx_v.at[li, pl.ds(0, bw)]],
                vals_v.at[li, pl.ds(0, bw)],
                gsem,
            )
            return carry

        lax.fori_loop(0, l, fire, 0)

        pltpu.make_async_copy(out_hbm.at[:, pl.ds(b0, bw)], vals_v, gsem).wait()

        cols = [pl.ds(g * _LANES, _LANES) for g in range(ngr)]

        def mx(li, ms):
            return tuple(
                jnp.maximum(ms[g], vals_v[li, cols[g]]) for g in range(ngr)
            )

        ms = lax.fori_loop(
            1, l, mx, tuple(vals_v[0, cols[g]] for g in range(ngr))
        )

        def es(li, ss):
            nxt = []
            for g in range(ngr):
                ev = jnp.exp(vals_v[li, cols[g]] - ms[g])
                vals_v[li, cols[g]] = ev
                nxt.append(ss[g] + ev)
            return tuple(nxt)

        zero = jnp.zeros((_LANES,), jnp.float32)
        ss = lax.fori_loop(0, l, es, (zero,) * ngr)
        invs = [1.0 / ss[g] for g in range(ngr)]

        def sc(li, carry):
            for g in range(ngr):
                vals_v[li, cols[g]] = vals_v[li, cols[g]] * invs[g]
            return carry

        lax.fori_loop(0, l, sc, 0)

        pltpu.sync_copy(vals_v, out_hbm.at[:, pl.ds(b0, bw)])

    return kern(feats_t, means)


def kernel(_paragraphs, features, table):
    b, _, l = features.shape
    means = _row_means(table)
    feats_t = features[:, 0, :].T
    out_t = _gather_softmax_t(feats_t, means, b, l)
    return out_t.T[:, None, :]


# speedup vs baseline: 2.9222x; 1.0828x over previous
import functools

import jax
import jax.numpy as jnp
from jax import lax
from jax.experimental import pallas as pl
from jax.experimental.pallas import tpu as pltpu
from jax.experimental.pallas import tpu_sc as plsc

_NC = 2
_NS = 16
_LANES = 16


def _row_means(table):
    v, e = table.shape
    rb = 32768
    t = rb // e
    grid = (v + rb - 1) // rb

    def body(x_ref, o_ref):
        x = x_ref[...].reshape(t, e, e)
        xt = jnp.transpose(x, (0, 2, 1))
        s = jnp.sum(xt, axis=1) * (1.0 / e)
        o_ref[...] = s.reshape(rb)

    out = pl.pallas_call(
        body,
        grid=(grid,),
        in_specs=[pl.BlockSpec((rb, e), lambda i: (i, 0))],
        out_specs=pl.BlockSpec((rb,), lambda i: (i,)),
        out_shape=jax.ShapeDtypeStruct((grid * rb,), jnp.float32),
    )(table)
    return out


def _gather_softmax_t(feats_t, means, b, l):
    nw = _NC * _NS
    bw = b // nw
    assert bw * nw == b
    assert bw <= 128
    ngr = bw // _LANES
    assert ngr * _LANES == bw

    mesh = plsc.VectorSubcoreMesh(core_axis_name="c", subcore_axis_name="s")
    n_means = means.shape[0]
    mchunk = n_means // _NS
    assert mchunk * _NS == n_means

    @functools.partial(
        pl.kernel,
        mesh=mesh,
        out_type=jax.ShapeDtypeStruct((l, b), jnp.float32),
        scratch_types=[
            pltpu.VMEM_SHARED((n_means,), jnp.float32),
            pltpu.VMEM((l, bw), jnp.int32),
            pltpu.VMEM((l, bw), jnp.float32),
            pltpu.SemaphoreType.DMA,
        ],
        compiler_params=pltpu.CompilerParams(
            use_tc_tiling_on_sc=False, needs_layout_passes=False
        ),
    )
    def kern(feat_hbm, means_hbm, out_hbm, means_sh, idx_v, vals_v, gsem):
        wid = lax.axis_index("s") * _NC + lax.axis_index("c")
        b0 = wid * bw
        sid = lax.axis_index("s")

        pltpu.sync_copy(
            means_hbm.at[pl.ds(sid * mchunk, mchunk)],
            means_sh.at[pl.ds(sid * mchunk, mchunk)],
        )
        pltpu.sync_copy(feat_hbm.at[:, pl.ds(b0, bw)], idx_v)
        plsc.subcore_barrier()

        def fire(li, carry):
            pltpu.async_copy(
                means_sh.at[idx_v.at[li, pl.ds(0, bw)]],
                vals_v.at[li, pl.ds(0, bw)],
                gsem,
            )
            return carry

        lax.fori_loop(0, l, fire, 0)

        pltpu.make_async_copy(out_hbm.at[:, pl.ds(b0, bw)], vals_v, gsem).wait()

        cols = [pl.ds(g * _LANES, _LANES) for g in range(ngr)]

        def mx(li, ms):
            return tuple(
                jnp.maximum(ms[g], vals_v[li, cols[g]]) for g in range(ngr)
            )

        ms = lax.fori_loop(
            1, l, mx, tuple(vals_v[0, cols[g]] for g in range(ngr))
        )

        def es(li, ss):
            nxt = []
            for g in range(ngr):
                ev = jnp.exp(vals_v[li, cols[g]] - ms[g])
                vals_v[li, cols[g]] = ev
                nxt.append(ss[g] + ev)
            return tuple(nxt)

        zero = jnp.zeros((_LANES,), jnp.float32)
        ss = lax.fori_loop(0, l, es, (zero,) * ngr)
        invs = [1.0 / ss[g] for g in range(ngr)]

        def sc(li, carry):
            for g in range(ngr):
                vals_v[li, cols[g]] = vals_v[li, cols[g]] * invs[g]
            return carry

        lax.fori_loop(0, l, sc, 0)

        pltpu.sync_copy(vals_v, out_hbm.at[:, pl.ds(b0, bw)])

    return kern(feats_t, means)


def kernel(_paragraphs, features, table):
    b, _, l = features.shape
    means = _row_means(table)
    feats_t = features[:, 0, :].T
    out_t = _gather_softmax_t(feats_t, means, b, l)
    return out_t.T[:, None, :]
